# Initial kernel scaffold; baseline (speedup 1.0000x reference)
import jax
import jax.numpy as jnp
from jax.experimental import pallas as pl


def kernel(x_new, x, y, T):
    raise NotImplementedError("write your pallas kernel here")


# speedup vs baseline: 1309.7682x; 1309.7682x over previous
import functools

import jax
import jax.numpy as jnp
from jax import lax
from jax.experimental import pallas as pl
from jax.experimental.pallas import tpu as pltpu
from jax.experimental.pallas import tpu_sc as plsc

L = 16
CHUNK = 8192


def _interp_kernel(B, K, N, NC, NS):
    NW = NC * NS
    wpr = NW // B
    qw = N // wpr
    n_chunks = qw // CHUNK
    eps = float(jnp.finfo(jnp.float32).eps)

    mesh = plsc.VectorSubcoreMesh(core_axis_name="c", subcore_axis_name="s")

    @functools.partial(
        pl.kernel,
        out_type=jax.ShapeDtypeStruct((B, N), jnp.float32),
        mesh=mesh,
        compiler_params=pltpu.CompilerParams(needs_layout_passes=False),
        scratch_types=[
            pltpu.VMEM((K,), jnp.float32),
            pltpu.VMEM((K,), jnp.float32),
            pltpu.VMEM((K,), jnp.float32),
            pltpu.VMEM((CHUNK,), jnp.float32),
            pltpu.VMEM((CHUNK,), jnp.float32),
        ],
    )
    def body(xq_hbm, x_hbm, y_hbm, out_hbm, xr, yr, sr, qb, ob):
        wid = lax.axis_index("s") * NC + lax.axis_index("c")
        row = wid // wpr
        qoff = (wid % wpr) * qw

        pltpu.sync_copy(x_hbm.at[row], xr)
        pltpu.sync_copy(y_hbm.at[row], yr)

        iota = lax.iota(jnp.int32, L)

        @plsc.parallel_loop(0, K, step=L, unroll=4)
        def _(base):
            idx1 = jnp.minimum(base + iota + 1, K - 1)
            x0 = xr[pl.ds(base, L)]
            y0 = yr[pl.ds(base, L)]
            x1 = plsc.load_gather(xr, [idx1])
            y1 = plsc.load_gather(yr, [idx1])
            sr[pl.ds(base, L)] = (y1 - y0) / (eps + (x1 - x0))

        def chunk_body(c, _):
            off = qoff + c * CHUNK
            pltpu.sync_copy(xq_hbm.at[row, pl.ds(off, CHUNK)], qb)

            @plsc.parallel_loop(0, CHUNK, step=L, unroll=4)
            def _(i):
                q = qb[pl.ds(i, L)]
                pos = jnp.zeros((L,), jnp.int32)
                s = K // 2
                while s >= 1:
                    xv = plsc.load_gather(xr, [pos + (s - 1)])
                    pos = jnp.where(xv < q, pos + s, pos)
                    s //= 2
                ind = jnp.clip(pos - 1, 0, K - 2)
                xv = plsc.load_gather(xr, [ind])
                yv = plsc.load_gather(yr, [ind])
                sv = plsc.load_gather(sr, [ind])
                ob[pl.ds(i, L)] = yv + sv * (q - xv)

            pltpu.sync_copy(ob, out_hbm.at[row, pl.ds(off, CHUNK)])
            return 0

        lax.fori_loop(0, n_chunks, chunk_body, 0)

    return body


@jax.jit
def kernel(x_new, x, y, T):
    del T
    B, N = x_new.shape
    K = x.shape[1]
    info = plsc.get_sparse_core_info()
    fn = _interp_kernel(B, K, N, info.num_cores, info.num_subcores)
    return fn(x_new, x, y)
